# baseline (speedup 1.0000x reference)
import functools

import jax
import jax.numpy as jnp
from jax import lax
from jax.experimental import pallas as pl
from jax.experimental.pallas import tpu as pltpu
from jax.experimental.pallas import tpu_sc as plsc

_SQRT_2_OVER_PI = 0.7978845608028654
_C = 64
_LANE_OF_HEAD = (0, 8, 4, 12, 2, 10, 6, 14)


def _proj_body(x_ref, wk_ref, bk_ref, wq_ref, bq_ref, wv_ref, bv_ref,
               k_ref, q_ref, v_ref):
    xb = x_ref[...]
    k_ref[...] = jnp.dot(xb, wk_ref[...], preferred_element_type=jnp.float32) + bk_ref[...]
    q_ref[...] = jnp.dot(xb, wq_ref[...], preferred_element_type=jnp.float32) + bq_ref[...]
    v_ref[...] = jnp.dot(xb, wv_ref[...], preferred_element_type=jnp.float32) + bv_ref[...]


def _projections(x, Wk2, bk2, Wq, bq, Wv2, bv2, tile, heads):
    n, d = x.shape
    grid = n // tile
    full = lambda i: (0, 0)
    row = lambda i: (i, 0)
    outs = pl.pallas_call(
        _proj_body,
        grid=(grid,),
        in_specs=[
            pl.BlockSpec((tile, d), row),
            pl.BlockSpec((d, d), full),
            pl.BlockSpec((1, d), full),
            pl.BlockSpec((d, d), full),
            pl.BlockSpec((1, d), full),
            pl.BlockSpec((d, d), full),
            pl.BlockSpec((1, d), full),
        ],
        out_specs=[pl.BlockSpec((tile, d), row)] * 3,
        out_shape=[jax.ShapeDtypeStruct((n, d), jnp.float32)] * 3,
    )(x, Wk2, bk2, Wq, bq, Wv2, bv2)
    return outs[0], outs[1], outs[2]


def _edge_body(n, e, heads, rows_per_tile, zr, iters, nchunk,
               k_hbm, q_hbm, src_hbm, dst_hbm, v_hbm,
               den_hbm, num_hbm, s_hbm,
               src_v, dst_v, k_v, q_v, vals_v, sc_v, z_v,
               acc_sh, sem1, sem2):
    vr_v = k_v

    cid = lax.axis_index("c")
    sid = lax.axis_index("s")
    w = sid * 2 + cid
    r0 = sid * rows_per_tile

    zero16 = jnp.zeros((16,), jnp.float32)
    idx16 = lax.iota(jnp.int32, 16)
    emask = jnp.where((idx16 & 1) == 0, 1.0, 0.0).astype(jnp.float32)

    def _bfly(v, kk):
        return v + v.at[idx16 ^ kk].get(mode='promise_in_bounds')

    def _zrow(i, _):
        z_v[i] = zero16
        return _
    lax.fori_loop(0, zr, _zrow, 0)

    def _zero_own_rows():
        def _zc(j, _):
            pltpu.sync_copy(z_v, acc_sh.at[pl.ds(r0 + j * zr, zr)])
            return _
        lax.fori_loop(0, rows_per_tile // zr, _zc, 0)

    _zero_own_rows()
    plsc.subcore_barrier()

    def _den_chunk(t, carry):
        g = w + t * 32

        @pl.when(g < nchunk)
        def _guarded():
            base = g * _C
            pltpu.sync_copy(src_hbm.at[pl.ds(base, _C)], src_v)
            pltpu.sync_copy(dst_hbm.at[pl.ds(base, _C)], dst_v)
            cp1 = pltpu.async_copy(k_hbm.at[src_v], k_v, sem1)
            cp2 = pltpu.async_copy(q_hbm.at[dst_v], q_v, sem2)
            cp1.wait()
            cp2.wait()

            def _alpha(i, _):
                l1 = [_bfly(k_v[i, pl.ds(h * 16, 16)] * q_v[i, pl.ds(h * 16, 16)], 8)
                      for h in range(heads)]
                m1 = [jnp.where(idx16 < 8, l1[2 * j], l1[2 * j + 1]) for j in range(4)]
                l2 = [_bfly(m, 4) for m in m1]
                m2 = [jnp.where((idx16 & 4) == 0, l2[2 * j], l2[2 * j + 1]) for j in range(2)]
                l3 = [_bfly(m, 2) for m in m2]
                m3 = jnp.where((idx16 & 2) == 0, l3[0], l3[1])
                f = _bfly(m3, 1)
                vals_v[i] = jnp.exp(f) * emask
                return _
            lax.fori_loop(0, _C, _alpha, 0)

            pltpu.sync_copy(vals_v, s_hbm.at[pl.ds(base, _C)])
            pltpu.sync_copy(vals_v, acc_sh.at[dst_v], add=True)
        return carry
    lax.fori_loop(0, iters, _den_chunk, 0)

    plsc.subcore_barrier()
    pltpu.sync_copy(acc_sh.at[pl.ds(r0, rows_per_tile)],
                    den_hbm.at[cid, pl.ds(r0, rows_per_tile)])

    for h in range(heads):
        _zero_own_rows()
        plsc.subcore_barrier()

        def _num_chunk(t, carry, h=h):
            g = w + t * 32

            @pl.when(g < nchunk)
            def _guarded():
                base = g * _C
                pltpu.sync_copy(src_hbm.at[pl.ds(base, _C)], src_v)
                pltpu.sync_copy(dst_hbm.at[pl.ds(base, _C)], dst_v)
                pltpu.sync_copy(s_hbm.at[pl.ds(base, _C)], vals_v)
                pltpu.async_copy(v_hbm.at[src_v], vr_v, sem1).wait()

                def _scale(i, _):
                    srow = vals_v[i]
                    sc_v[i] = vr_v[i, pl.ds(h * 16, 16)] * srow[_LANE_OF_HEAD[h]]
                    return _
                lax.fori_loop(0, _C, _scale, 0)

                pltpu.sync_copy(sc_v, acc_sh.at[dst_v], add=True)
            return carry
        lax.fori_loop(0, iters, _num_chunk, 0)

        plsc.subcore_barrier()
        pltpu.sync_copy(acc_sh.at[pl.ds(r0, rows_per_tile)],
                        num_hbm.at[cid * heads + h, pl.ds(r0, rows_per_tile)])
        plsc.subcore_barrier()


def _edge_phase(k, q, v, src, dst, n, e, heads):
    nchunk = e // _C
    iters = (nchunk + 31) // 32
    rows_per_tile = -(-(n // 16) // 8) * 8
    n_pad = rows_per_tile * 16
    zr = 368
    assert rows_per_tile % zr == 0
    mesh = plsc.VectorSubcoreMesh(core_axis_name="c", subcore_axis_name="s")
    body = functools.partial(_edge_body, n_pad, e, heads, rows_per_tile, zr, iters, nchunk)
    f = pl.kernel(
        body,
        mesh=mesh,
        out_type=[
            jax.ShapeDtypeStruct((2, n_pad, 16), jnp.float32),
            jax.ShapeDtypeStruct((2 * heads, n_pad, 16), jnp.float32),
            jax.ShapeDtypeStruct((e, 16), jnp.float32),
        ],
        scratch_types=[
            pltpu.VMEM((_C,), jnp.int32),
            pltpu.VMEM((_C,), jnp.int32),
            pltpu.VMEM((_C, 128), jnp.float32),
            pltpu.VMEM((_C, 128), jnp.float32),
            pltpu.VMEM((_C, 16), jnp.float32),
            pltpu.VMEM((_C, 16), jnp.float32),
            pltpu.VMEM((zr, 16), jnp.float32),
            pltpu.VMEM_SHARED((n_pad, 16), jnp.float32),
            pltpu.SemaphoreType.DMA,
            pltpu.SemaphoreType.DMA,
        ],
        compiler_params=pltpu.CompilerParams(use_tc_tiling_on_sc=False),
    )
    den, num, _s = f(k, q, src, dst, v)
    return den, num


def _post_body(num_ref, den_ref, x_ref, pos_ref, wa_ref, ba_ref, wn_ref,
               bn_ref, beta_ref, lng_ref, lnb_ref, o_ref):
    heads = num_ref.shape[0] // 2
    t = x_ref.shape[0]
    d = x_ref.shape[1]
    dh = d // heads
    num = num_ref[...]
    den = den_ref[...]
    nsum = num[:heads] + num[heads:]
    dsum = den[0] + den[1]
    aggh = jnp.concatenate([nsum[h] for h in range(heads)], axis=-1)
    drep = jnp.concatenate(
        [jnp.broadcast_to(dsum[:, l:l + 1], (t, dh)) for l in _LANE_OF_HEAD[:heads]],
        axis=-1)
    agg = aggh / (drep + 1e-16)
    g = 0.5 * agg * (1.0 + jnp.tanh(_SQRT_2_OVER_PI * (agg + 0.044715 * agg * agg * agg)))
    o = jnp.dot(g, wa_ref[...], preferred_element_type=jnp.float32) + ba_ref[...]
    beta = beta_ref[0, 0]
    o = beta * o + (1.0 - beta) * x_ref[...]
    hh = jnp.dot(o, wn_ref[...], preferred_element_type=jnp.float32) + bn_ref[...] + pos_ref[...]
    mu = jnp.mean(hh, axis=-1, keepdims=True)
    var = jnp.mean((hh - mu) ** 2, axis=-1, keepdims=True)
    o_ref[...] = (hh - mu) * jax.lax.rsqrt(var + 1e-12) * lng_ref[...] + lnb_ref[...]


def _post_stage(num, den, x, pos_table, Wa, ba, Wn, bn, beta, ln_g, ln_b, tile):
    n, d = x.shape
    hh2 = num.shape[0]
    grid = n // tile
    full = lambda i: (0, 0)
    row = lambda i: (i, 0)
    row3 = lambda i: (0, i, 0)
    return pl.pallas_call(
        _post_body,
        grid=(grid,),
        in_specs=[
            pl.BlockSpec((hh2, tile, 16), row3),
            pl.BlockSpec((2, tile, 16), row3),
            pl.BlockSpec((tile, d), row),
            pl.BlockSpec((tile, d), row),
            pl.BlockSpec((d, d), full),
            pl.BlockSpec((1, d), full),
            pl.BlockSpec((d, d), full),
            pl.BlockSpec((1, d), full),
            pl.BlockSpec((1, 1), full),
            pl.BlockSpec((1, d), full),
            pl.BlockSpec((1, d), full),
        ],
        out_specs=pl.BlockSpec((tile, d), row),
        out_shape=jax.ShapeDtypeStruct((n, d), jnp.float32),
    )(num, den, x, pos_table, Wa, ba, Wn, bn, beta, ln_g, ln_b)


def kernel(x, edge_index, Wk, bk, Wq, bq, Wv, bv, a_rel, m_rel, p_rel,
           Wa, ba, skip, Wn, bn, pos_table, ln_g, ln_b):
    n, d = x.shape
    heads, dh = a_rel.shape[0], a_rel.shape[1]
    e = edge_index.shape[1]

    scale = p_rel / jnp.sqrt(jnp.float32(dh))
    Wk2 = jnp.einsum('ihd,hde->ihe', Wk.reshape(d, heads, dh), a_rel)
    Wk2 = (Wk2 * scale[None, :, None]).reshape(d, d)
    bk2 = (jnp.einsum('hd,hde->he', bk.reshape(heads, dh), a_rel) * scale[:, None]).reshape(1, d)
    Wv2 = jnp.einsum('ihd,hde->ihe', Wv.reshape(d, heads, dh), m_rel).reshape(d, d)
    bv2 = jnp.einsum('hd,hde->he', bv.reshape(heads, dh), m_rel).reshape(1, d)

    k, q, v = _projections(x, Wk2, bk2, Wq, bq.reshape(1, d), Wv2, bv2,
                           tile=2000, heads=heads)

    src = edge_index[0].astype(jnp.int32)
    dst = edge_index[1].astype(jnp.int32)
    den, num = _edge_phase(k, q, v, src, dst, n, e, heads)

    beta = jax.nn.sigmoid(skip).reshape(1, 1)
    return _post_stage(num, den, x, pos_table, Wa, ba.reshape(1, d),
                       Wn, bn.reshape(1, d), beta, ln_g.reshape(1, d),
                       ln_b.reshape(1, d), tile=2000)


# speedup vs baseline: 1.3803x; 1.3803x over previous
import functools

import jax
import jax.numpy as jnp
from jax import lax
from jax.experimental import pallas as pl
from jax.experimental.pallas import tpu as pltpu
from jax.experimental.pallas import tpu_sc as plsc

_SQRT_2_OVER_PI = 0.7978845608028654
_C = 64
_CN = 128
_LANE_OF_HEAD = (0, 8, 4, 12, 2, 10, 6, 14)


def _proj_body(x_ref, wk_ref, bk_ref, wq_ref, bq_ref, wv_ref, bv_ref,
               k_ref, q_ref, v_ref):
    xb = x_ref[...]
    k_ref[...] = jnp.dot(xb, wk_ref[...], preferred_element_type=jnp.float32) + bk_ref[...]
    q_ref[...] = jnp.dot(xb, wq_ref[...], preferred_element_type=jnp.float32) + bq_ref[...]
    v_ref[...] = jnp.dot(xb, wv_ref[...], preferred_element_type=jnp.float32) + bv_ref[...]


def _projections(x, Wk2, bk2, Wq, bq, Wv2, bv2, tile, heads):
    n, d = x.shape
    grid = n // tile
    full = lambda i: (0, 0)
    row = lambda i: (i, 0)
    outs = pl.pallas_call(
        _proj_body,
        grid=(grid,),
        in_specs=[
            pl.BlockSpec((tile, d), row),
            pl.BlockSpec((d, d), full),
            pl.BlockSpec((1, d), full),
            pl.BlockSpec((d, d), full),
            pl.BlockSpec((1, d), full),
            pl.BlockSpec((d, d), full),
            pl.BlockSpec((1, d), full),
        ],
        out_specs=[pl.BlockSpec((tile, d), row)] * 3,
        out_shape=[jax.ShapeDtypeStruct((n, d), jnp.float32)] * 3,
    )(x, Wk2, bk2, Wq, bq, Wv2, bv2)
    return outs[0], outs[1], outs[2]


def _edge_body(n, e, heads, rows_per_tile, zr, iters_d, nchunk_d, iters_n, nchunk_n,
               k_hbm, q_hbm, src_hbm, dst_hbm, v_hbm,
               den_hbm, num_hbm, s_hbm,
               src64_v, dst64_v, src128_v, dst128_v, rows_v, vals64_v, vals128_v, sc_v, z_v,
               acc_sh, sem1, sem2):

    cid = lax.axis_index("c")
    sid = lax.axis_index("s")
    w = sid * 2 + cid
    r0 = sid * rows_per_tile

    zero16 = jnp.zeros((16,), jnp.float32)
    idx16 = lax.iota(jnp.int32, 16)
    emask = jnp.where((idx16 & 1) == 0, 1.0, 0.0).astype(jnp.float32)

    def _bfly(v, kk):
        return v + v.at[idx16 ^ kk].get(mode='promise_in_bounds')

    def _zrow(i, _):
        z_v[i] = zero16
        return _
    lax.fori_loop(0, zr, _zrow, 0)

    def _zero_own_rows():
        def _zc(j, _):
            pltpu.sync_copy(z_v, acc_sh.at[pl.ds(r0 + j * zr, zr)])
            return _
        lax.fori_loop(0, rows_per_tile // zr, _zc, 0)

    _zero_own_rows()
    plsc.subcore_barrier()

    def _den_chunk(t, carry):
        g = w + t * 32

        @pl.when(g < nchunk_d)
        def _guarded():
            base = g * _C
            pltpu.sync_copy(src_hbm.at[pl.ds(base, _C)], src64_v)
            pltpu.sync_copy(dst_hbm.at[pl.ds(base, _C)], dst64_v)
            cp1 = pltpu.async_copy(k_hbm.at[src64_v], rows_v.at[pl.ds(0, _C)], sem1)
            cp2 = pltpu.async_copy(q_hbm.at[dst64_v], rows_v.at[pl.ds(_C, _C)], sem2)
            cp1.wait()
            cp2.wait()

            def _alpha(i, _):
                l1 = [_bfly(rows_v[i, pl.ds(h * 16, 16)]
                            * rows_v[_C + i, pl.ds(h * 16, 16)], 8)
                      for h in range(heads)]
                m1 = [jnp.where(idx16 < 8, l1[2 * j], l1[2 * j + 1]) for j in range(4)]
                l2 = [_bfly(m, 4) for m in m1]
                m2 = [jnp.where((idx16 & 4) == 0, l2[2 * j], l2[2 * j + 1]) for j in range(2)]
                l3 = [_bfly(m, 2) for m in m2]
                m3 = jnp.where((idx16 & 2) == 0, l3[0], l3[1])
                f = _bfly(m3, 1)
                vals64_v[i] = jnp.exp(f) * emask
                return _
            lax.fori_loop(0, _C, _alpha, 0)

            pltpu.sync_copy(vals64_v, s_hbm.at[pl.ds(base, _C)])
            pltpu.sync_copy(vals64_v, acc_sh.at[dst64_v], add=True)
        return carry
    lax.fori_loop(0, iters_d, _den_chunk, 0)

    plsc.subcore_barrier()
    pltpu.sync_copy(acc_sh.at[pl.ds(r0, rows_per_tile)],
                    den_hbm.at[cid, pl.ds(r0, rows_per_tile)])

    for h in range(heads):
        _zero_own_rows()
        plsc.subcore_barrier()

        def _num_chunk(t, carry, h=h):
            g = w + t * 32

            @pl.when(g < nchunk_n)
            def _guarded():
                base = g * _CN
                pltpu.sync_copy(src_hbm.at[pl.ds(base, _CN)], src128_v)
                pltpu.sync_copy(dst_hbm.at[pl.ds(base, _CN)], dst128_v)
                pltpu.sync_copy(s_hbm.at[pl.ds(base, _CN)], vals128_v)
                pltpu.async_copy(v_hbm.at[src128_v], rows_v, sem1).wait()

                def _scale(i, _):
                    srow = vals128_v[i]
                    sc_v[i] = rows_v[i, pl.ds(h * 16, 16)] * srow[_LANE_OF_HEAD[h]]
                    return _
                lax.fori_loop(0, _CN, _scale, 0)

                pltpu.sync_copy(sc_v, acc_sh.at[dst128_v], add=True)
            return carry
        lax.fori_loop(0, iters_n, _num_chunk, 0)

        plsc.subcore_barrier()
        pltpu.sync_copy(acc_sh.at[pl.ds(r0, rows_per_tile)],
                        num_hbm.at[cid * heads + h, pl.ds(r0, rows_per_tile)])
        plsc.subcore_barrier()


def _edge_phase(k, q, v, src, dst, n, e, heads):
    nchunk_d = e // _C
    iters_d = (nchunk_d + 31) // 32
    nchunk_n = e // _CN
    iters_n = (nchunk_n + 31) // 32
    rows_per_tile = -(-(n // 16) // 8) * 8
    n_pad = rows_per_tile * 16
    zr = 368
    assert rows_per_tile % zr == 0
    mesh = plsc.VectorSubcoreMesh(core_axis_name="c", subcore_axis_name="s")
    body = functools.partial(_edge_body, n_pad, e, heads, rows_per_tile, zr,
                             iters_d, nchunk_d, iters_n, nchunk_n)
    f = pl.kernel(
        body,
        mesh=mesh,
        out_type=[
            jax.ShapeDtypeStruct((2, n_pad, 16), jnp.float32),
            jax.ShapeDtypeStruct((2 * heads, n_pad, 16), jnp.float32),
            jax.ShapeDtypeStruct((e, 16), jnp.float32),
        ],
        scratch_types=[
            pltpu.VMEM((_C,), jnp.int32),
            pltpu.VMEM((_C,), jnp.int32),
            pltpu.VMEM((_CN,), jnp.int32),
            pltpu.VMEM((_CN,), jnp.int32),
            pltpu.VMEM((_CN, 128), jnp.float32),
            pltpu.VMEM((_C, 16), jnp.float32),
            pltpu.VMEM((_CN, 16), jnp.float32),
            pltpu.VMEM((_CN, 16), jnp.float32),
            pltpu.VMEM((zr, 16), jnp.float32),
            pltpu.VMEM_SHARED((n_pad, 16), jnp.float32),
            pltpu.SemaphoreType.DMA,
            pltpu.SemaphoreType.DMA,
        ],
        compiler_params=pltpu.CompilerParams(use_tc_tiling_on_sc=False),
    )
    den, num, _s = f(k, q, src, dst, v)
    return den, num


def _post_body(num_ref, den_ref, x_ref, pos_ref, wa_ref, ba_ref, wn_ref,
               bn_ref, beta_ref, lng_ref, lnb_ref, o_ref):
    heads = num_ref.shape[0] // 2
    t = x_ref.shape[0]
    d = x_ref.shape[1]
    dh = d // heads
    num = num_ref[...]
    den = den_ref[...]
    nsum = num[:heads] + num[heads:]
    dsum = den[0] + den[1]
    aggh = jnp.concatenate([nsum[h] for h in range(heads)], axis=-1)
    drep = jnp.concatenate(
        [jnp.broadcast_to(dsum[:, l:l + 1], (t, dh)) for l in _LANE_OF_HEAD[:heads]],
        axis=-1)
    agg = aggh / (drep + 1e-16)
    g = 0.5 * agg * (1.0 + jnp.tanh(_SQRT_2_OVER_PI * (agg + 0.044715 * agg * agg * agg)))
    o = jnp.dot(g, wa_ref[...], preferred_element_type=jnp.float32) + ba_ref[...]
    beta = beta_ref[0, 0]
    o = beta * o + (1.0 - beta) * x_ref[...]
    hh = jnp.dot(o, wn_ref[...], preferred_element_type=jnp.float32) + bn_ref[...] + pos_ref[...]
    mu = jnp.mean(hh, axis=-1, keepdims=True)
    var = jnp.mean((hh - mu) ** 2, axis=-1, keepdims=True)
    o_ref[...] = (hh - mu) * jax.lax.rsqrt(var + 1e-12) * lng_ref[...] + lnb_ref[...]


def _post_stage(num, den, x, pos_table, Wa, ba, Wn, bn, beta, ln_g, ln_b, tile):
    n, d = x.shape
    hh2 = num.shape[0]
    grid = n // tile
    full = lambda i: (0, 0)
    row = lambda i: (i, 0)
    row3 = lambda i: (0, i, 0)
    return pl.pallas_call(
        _post_body,
        grid=(grid,),
        in_specs=[
            pl.BlockSpec((hh2, tile, 16), row3),
            pl.BlockSpec((2, tile, 16), row3),
            pl.BlockSpec((tile, d), row),
            pl.BlockSpec((tile, d), row),
            pl.BlockSpec((d, d), full),
            pl.BlockSpec((1, d), full),
            pl.BlockSpec((d, d), full),
            pl.BlockSpec((1, d), full),
            pl.BlockSpec((1, 1), full),
            pl.BlockSpec((1, d), full),
            pl.BlockSpec((1, d), full),
        ],
        out_specs=pl.BlockSpec((tile, d), row),
        out_shape=jax.ShapeDtypeStruct((n, d), jnp.float32),
    )(num, den, x, pos_table, Wa, ba, Wn, bn, beta, ln_g, ln_b)


def kernel(x, edge_index, Wk, bk, Wq, bq, Wv, bv, a_rel, m_rel, p_rel,
           Wa, ba, skip, Wn, bn, pos_table, ln_g, ln_b):
    n, d = x.shape
    heads, dh = a_rel.shape[0], a_rel.shape[1]
    e = edge_index.shape[1]

    scale = p_rel / jnp.sqrt(jnp.float32(dh))
    Wk2 = jnp.einsum('ihd,hde->ihe', Wk.reshape(d, heads, dh), a_rel)
    Wk2 = (Wk2 * scale[None, :, None]).reshape(d, d)
    bk2 = (jnp.einsum('hd,hde->he', bk.reshape(heads, dh), a_rel) * scale[:, None]).reshape(1, d)
    Wv2 = jnp.einsum('ihd,hde->ihe', Wv.reshape(d, heads, dh), m_rel).reshape(d, d)
    bv2 = jnp.einsum('hd,hde->he', bv.reshape(heads, dh), m_rel).reshape(1, d)

    k, q, v = _projections(x, Wk2, bk2, Wq, bq.reshape(1, d), Wv2, bv2,
                           tile=2000, heads=heads)

    src = edge_index[0].astype(jnp.int32)
    dst = edge_index[1].astype(jnp.int32)
    den, num = _edge_phase(k, q, v, src, dst, n, e, heads)

    beta = jax.nn.sigmoid(skip).reshape(1, 1)
    return _post_stage(num, den, x, pos_table, Wa, ba.reshape(1, d),
                       Wn, bn.reshape(1, d), beta, ln_g.reshape(1, d),
                       ln_b.reshape(1, d), tile=2000)


# speedup vs baseline: 1.7951x; 1.3005x over previous
import functools

import jax
import jax.numpy as jnp
from jax import lax
from jax.experimental import pallas as pl
from jax.experimental.pallas import tpu as pltpu
from jax.experimental.pallas import tpu_sc as plsc

_SQRT_2_OVER_PI = 0.7978845608028654
_C = 64
_CN = 128
_LANE_OF_HEAD = (0, 8, 4, 12, 2, 10, 6, 14)


def _proj_body(x_ref, wk_ref, bk_ref, wq_ref, bq_ref, wv_ref, bv_ref,
               k_ref, q_ref, v_ref):
    xb = x_ref[...]
    k_ref[...] = jnp.dot(xb, wk_ref[...], preferred_element_type=jnp.float32) + bk_ref[...]
    q_ref[...] = jnp.dot(xb, wq_ref[...], preferred_element_type=jnp.float32) + bq_ref[...]
    v_ref[...] = jnp.dot(xb, wv_ref[...], preferred_element_type=jnp.float32) + bv_ref[...]


def _projections(x, Wk2, bk2, Wq, bq, Wv2, bv2, tile, heads):
    n, d = x.shape
    grid = n // tile
    full = lambda i: (0, 0)
    row = lambda i: (i, 0)
    outs = pl.pallas_call(
        _proj_body,
        grid=(grid,),
        in_specs=[
            pl.BlockSpec((tile, d), row),
            pl.BlockSpec((d, d), full),
            pl.BlockSpec((1, d), full),
            pl.BlockSpec((d, d), full),
            pl.BlockSpec((1, d), full),
            pl.BlockSpec((d, d), full),
            pl.BlockSpec((1, d), full),
        ],
        out_specs=[pl.BlockSpec((tile, d), row)] * 3,
        out_shape=[jax.ShapeDtypeStruct((n, d), jnp.float32)] * 3,
    )(x, Wk2, bk2, Wq, bq, Wv2, bv2)
    return outs[0], outs[1], outs[2]


def _edge_body(n, e, heads, rows_per_tile, zr, iters_d, nchunk_d, iters_n, nchunk_n,
               k_hbm, q_hbm, src_hbm, dst_hbm, v_hbm,
               den_hbm, num_hbm, s_hbm,
               src64_v, dst64_v, src128_v, dst128_v, rows_v, vals64_v, vals128_v, sc_v, z_v,
               acc_sh, sem1, sem2):

    cid = lax.axis_index("c")
    sid = lax.axis_index("s")
    w = sid * 2 + cid
    r0 = sid * rows_per_tile

    zero16 = jnp.zeros((16,), jnp.float32)
    idx16 = lax.iota(jnp.int32, 16)
    emask = jnp.where((idx16 & 1) == 0, 1.0, 0.0).astype(jnp.float32)

    def _bfly(v, kk):
        return v + v.at[idx16 ^ kk].get(mode='promise_in_bounds')

    def _zrow(i, _):
        z_v[i] = zero16
        return _
    lax.fori_loop(0, zr, _zrow, 0)

    def _zero_own_rows():
        def _zc(j, _):
            pltpu.sync_copy(z_v, acc_sh.at[pl.ds(r0 + j * zr, zr)])
            return _
        lax.fori_loop(0, rows_per_tile // zr, _zc, 0)

    _zero_own_rows()
    plsc.subcore_barrier()

    def _den_chunk(t, carry):
        g = w + t * 32

        @pl.when(g < nchunk_d)
        def _guarded():
            base = g * _C
            cps = pltpu.async_copy(src_hbm.at[pl.ds(base, _C)], src64_v, sem1)
            cpd = pltpu.async_copy(dst_hbm.at[pl.ds(base, _C)], dst64_v, sem2)
            cps.wait()
            cp1 = pltpu.async_copy(k_hbm.at[src64_v], rows_v.at[pl.ds(0, _C)], sem1)
            cpd.wait()
            cp2 = pltpu.async_copy(q_hbm.at[dst64_v], rows_v.at[pl.ds(_C, _C)], sem2)
            cp1.wait()
            cp2.wait()

            def _alpha(i, _):
                l1 = [_bfly(rows_v[i, pl.ds(h * 16, 16)]
                            * rows_v[_C + i, pl.ds(h * 16, 16)], 8)
                      for h in range(heads)]
                m1 = [jnp.where(idx16 < 8, l1[2 * j], l1[2 * j + 1]) for j in range(4)]
                l2 = [_bfly(m, 4) for m in m1]
                m2 = [jnp.where((idx16 & 4) == 0, l2[2 * j], l2[2 * j + 1]) for j in range(2)]
                l3 = [_bfly(m, 2) for m in m2]
                m3 = jnp.where((idx16 & 2) == 0, l3[0], l3[1])
                f = _bfly(m3, 1)
                vals64_v[i] = jnp.exp(f) * emask
                return _
            lax.fori_loop(0, _C, _alpha, 0)

            pltpu.sync_copy(vals64_v, s_hbm.at[pl.ds(base, _C)])
            pltpu.sync_copy(vals64_v, acc_sh.at[dst64_v], add=True)
        return carry
    lax.fori_loop(0, iters_d, _den_chunk, 0)

    plsc.subcore_barrier()
    pltpu.sync_copy(acc_sh.at[pl.ds(r0, rows_per_tile)],
                    den_hbm.at[cid, pl.ds(r0, rows_per_tile)])

    for h in range(heads):
        _zero_own_rows()
        plsc.subcore_barrier()

        def _num_chunk(t, carry, h=h):
            g = w + t * 32

            @pl.when(g < nchunk_n)
            def _guarded():
                base = g * _CN
                cps = pltpu.async_copy(src_hbm.at[pl.ds(base, _CN)], src128_v, sem1)
                cpd = pltpu.async_copy(dst_hbm.at[pl.ds(base, _CN)], dst128_v, sem2)
                cpv = pltpu.async_copy(s_hbm.at[pl.ds(base, _CN)], vals128_v, sem2)
                cps.wait()
                cpg = pltpu.async_copy(v_hbm.at[src128_v], rows_v, sem1)
                cpd.wait()
                cpv.wait()
                cpg.wait()

                def _scale(i, _):
                    srow = vals128_v[i]
                    sc_v[i] = rows_v[i, pl.ds(h * 16, 16)] * srow[_LANE_OF_HEAD[h]]
                    return _
                lax.fori_loop(0, _CN, _scale, 0)

                pltpu.sync_copy(sc_v, acc_sh.at[dst128_v], add=True)
            return carry
        lax.fori_loop(0, iters_n, _num_chunk, 0)

        plsc.subcore_barrier()
        pltpu.sync_copy(acc_sh.at[pl.ds(r0, rows_per_tile)],
                        num_hbm.at[cid * heads + h, pl.ds(r0, rows_per_tile)])
        plsc.subcore_barrier()


def _edge_phase(k, q, v, src, dst, n, e, heads):
    nchunk_d = e // _C
    iters_d = (nchunk_d + 31) // 32
    nchunk_n = e // _CN
    iters_n = (nchunk_n + 31) // 32
    rows_per_tile = -(-(n // 16) // 8) * 8
    n_pad = rows_per_tile * 16
    zr = 368
    assert rows_per_tile % zr == 0
    mesh = plsc.VectorSubcoreMesh(core_axis_name="c", subcore_axis_name="s")
    body = functools.partial(_edge_body, n_pad, e, heads, rows_per_tile, zr,
                             iters_d, nchunk_d, iters_n, nchunk_n)
    f = pl.kernel(
        body,
        mesh=mesh,
        out_type=[
            jax.ShapeDtypeStruct((2, n_pad, 16), jnp.float32),
            jax.ShapeDtypeStruct((2 * heads, n_pad, 16), jnp.float32),
            jax.ShapeDtypeStruct((e, 16), jnp.float32),
        ],
        scratch_types=[
            pltpu.VMEM((_C,), jnp.int32),
            pltpu.VMEM((_C,), jnp.int32),
            pltpu.VMEM((_CN,), jnp.int32),
            pltpu.VMEM((_CN,), jnp.int32),
            pltpu.VMEM((_CN, 128), jnp.float32),
            pltpu.VMEM((_C, 16), jnp.float32),
            pltpu.VMEM((_CN, 16), jnp.float32),
            pltpu.VMEM((_CN, 16), jnp.float32),
            pltpu.VMEM((zr, 16), jnp.float32),
            pltpu.VMEM_SHARED((n_pad, 16), jnp.float32),
            pltpu.SemaphoreType.DMA,
            pltpu.SemaphoreType.DMA,
        ],
        compiler_params=pltpu.CompilerParams(use_tc_tiling_on_sc=False),
    )
    den, num, _s = f(k, q, src, dst, v)
    return den, num


def _post_body(num_ref, den_ref, x_ref, pos_ref, wa_ref, ba_ref, wn_ref,
               bn_ref, beta_ref, lng_ref, lnb_ref, o_ref):
    heads = num_ref.shape[0] // 2
    t = x_ref.shape[0]
    d = x_ref.shape[1]
    dh = d // heads
    num = num_ref[...]
    den = den_ref[...]
    nsum = num[:heads] + num[heads:]
    dsum = den[0] + den[1]
    aggh = jnp.concatenate([nsum[h] for h in range(heads)], axis=-1)
    drep = jnp.concatenate(
        [jnp.broadcast_to(dsum[:, l:l + 1], (t, dh)) for l in _LANE_OF_HEAD[:heads]],
        axis=-1)
    agg = aggh / (drep + 1e-16)
    g = 0.5 * agg * (1.0 + jnp.tanh(_SQRT_2_OVER_PI * (agg + 0.044715 * agg * agg * agg)))
    o = jnp.dot(g, wa_ref[...], preferred_element_type=jnp.float32) + ba_ref[...]
    beta = beta_ref[0, 0]
    o = beta * o + (1.0 - beta) * x_ref[...]
    hh = jnp.dot(o, wn_ref[...], preferred_element_type=jnp.float32) + bn_ref[...] + pos_ref[...]
    mu = jnp.mean(hh, axis=-1, keepdims=True)
    var = jnp.mean((hh - mu) ** 2, axis=-1, keepdims=True)
    o_ref[...] = (hh - mu) * jax.lax.rsqrt(var + 1e-12) * lng_ref[...] + lnb_ref[...]


def _post_stage(num, den, x, pos_table, Wa, ba, Wn, bn, beta, ln_g, ln_b, tile):
    n, d = x.shape
    hh2 = num.shape[0]
    grid = n // tile
    full = lambda i: (0, 0)
    row = lambda i: (i, 0)
    row3 = lambda i: (0, i, 0)
    return pl.pallas_call(
        _post_body,
        grid=(grid,),
        in_specs=[
            pl.BlockSpec((hh2, tile, 16), row3),
            pl.BlockSpec((2, tile, 16), row3),
            pl.BlockSpec((tile, d), row),
            pl.BlockSpec((tile, d), row),
            pl.BlockSpec((d, d), full),
            pl.BlockSpec((1, d), full),
            pl.BlockSpec((d, d), full),
            pl.BlockSpec((1, d), full),
            pl.BlockSpec((1, 1), full),
            pl.BlockSpec((1, d), full),
            pl.BlockSpec((1, d), full),
        ],
        out_specs=pl.BlockSpec((tile, d), row),
        out_shape=jax.ShapeDtypeStruct((n, d), jnp.float32),
    )(num, den, x, pos_table, Wa, ba, Wn, bn, beta, ln_g, ln_b)


def kernel(x, edge_index, Wk, bk, Wq, bq, Wv, bv, a_rel, m_rel, p_rel,
           Wa, ba, skip, Wn, bn, pos_table, ln_g, ln_b):
    n, d = x.shape
    heads, dh = a_rel.shape[0], a_rel.shape[1]
    e = edge_index.shape[1]

    scale = p_rel / jnp.sqrt(jnp.float32(dh))
    Wk2 = jnp.einsum('ihd,hde->ihe', Wk.reshape(d, heads, dh), a_rel)
    Wk2 = (Wk2 * scale[None, :, None]).reshape(d, d)
    bk2 = (jnp.einsum('hd,hde->he', bk.reshape(heads, dh), a_rel) * scale[:, None]).reshape(1, d)
    Wv2 = jnp.einsum('ihd,hde->ihe', Wv.reshape(d, heads, dh), m_rel).reshape(d, d)
    bv2 = jnp.einsum('hd,hde->he', bv.reshape(heads, dh), m_rel).reshape(1, d)

    k, q, v = _projections(x, Wk2, bk2, Wq, bq.reshape(1, d), Wv2, bv2,
                           tile=2000, heads=heads)

    src = edge_index[0].astype(jnp.int32)
    dst = edge_index[1].astype(jnp.int32)
    den, num = _edge_phase(k, q, v, src, dst, n, e, heads)

    beta = jax.nn.sigmoid(skip).reshape(1, 1)
    return _post_stage(num, den, x, pos_table, Wa, ba.reshape(1, d),
                       Wn, bn.reshape(1, d), beta, ln_g.reshape(1, d),
                       ln_b.reshape(1, d), tile=2000)


# speedup vs baseline: 1.8057x; 1.0059x over previous
import functools

import jax
import jax.numpy as jnp
from jax import lax
from jax.experimental import pallas as pl
from jax.experimental.pallas import tpu as pltpu
from jax.experimental.pallas import tpu_sc as plsc

_SQRT_2_OVER_PI = 0.7978845608028654
_C = 64
_CN = 128
_LANE_OF_HEAD = (0, 8, 4, 12, 2, 10, 6, 14)


def _proj_body(x_ref, wk_ref, bk_ref, wq_ref, bq_ref, wv_ref, bv_ref,
               k_ref, q_ref, v_ref):
    xb = x_ref[...]
    k_ref[...] = jnp.dot(xb, wk_ref[...], preferred_element_type=jnp.float32) + bk_ref[...]
    q_ref[...] = jnp.dot(xb, wq_ref[...], preferred_element_type=jnp.float32) + bq_ref[...]
    v_ref[...] = jnp.dot(xb, wv_ref[...], preferred_element_type=jnp.float32) + bv_ref[...]


def _projections(x, Wk2, bk2, Wq, bq, Wv2, bv2, tile, heads):
    n, d = x.shape
    grid = n // tile
    full = lambda i: (0, 0)
    row = lambda i: (i, 0)
    outs = pl.pallas_call(
        _proj_body,
        grid=(grid,),
        in_specs=[
            pl.BlockSpec((tile, d), row),
            pl.BlockSpec((d, d), full),
            pl.BlockSpec((1, d), full),
            pl.BlockSpec((d, d), full),
            pl.BlockSpec((1, d), full),
            pl.BlockSpec((d, d), full),
            pl.BlockSpec((1, d), full),
        ],
        out_specs=[pl.BlockSpec((tile, d), row)] * 3,
        out_shape=[jax.ShapeDtypeStruct((n, d), jnp.float32)] * 3,
    )(x, Wk2, bk2, Wq, bq, Wv2, bv2)
    return outs[0], outs[1], outs[2]


def _edge_body(n, e, heads, rows_per_tile, zr, iters_d, nchunk_d, iters_n, nchunk_n,
               k_hbm, q_hbm, src_hbm, dst_hbm, v_hbm,
               den_hbm, num_hbm, s_hbm,
               src64_v, dst64_v, src128_v, dst128_v, rows_v, vals64_v, vals128_v, sc_v, z_v,
               acc_sh, sem1, sem2):

    cid = lax.axis_index("c")
    sid = lax.axis_index("s")
    w = sid * 2 + cid
    r0 = sid * rows_per_tile

    zero16 = jnp.zeros((16,), jnp.float32)
    idx16 = lax.iota(jnp.int32, 16)
    emask = jnp.where((idx16 & 1) == 0, 1.0, 0.0).astype(jnp.float32)

    def _bfly(v, kk):
        return v + v.at[idx16 ^ kk].get(mode='promise_in_bounds')

    def _zrow(i, _):
        z_v[i] = zero16
        return _
    lax.fori_loop(0, zr, _zrow, 0)

    def _zero_own_rows():
        def _zc(j, _):
            pltpu.sync_copy(z_v, acc_sh.at[pl.ds(r0 + j * zr, zr)])
            return _
        lax.fori_loop(0, rows_per_tile // zr, _zc, 0)

    _zero_own_rows()
    plsc.subcore_barrier()

    def _den_chunk(t, carry):
        g = w + t * 32

        @pl.when(g < nchunk_d)
        def _guarded():
            base = g * _C
            cps = pltpu.async_copy(src_hbm.at[pl.ds(base, _C)], src64_v, sem1)
            cpd = pltpu.async_copy(dst_hbm.at[pl.ds(base, _C)], dst64_v, sem2)
            cps.wait()
            cp1 = pltpu.async_copy(k_hbm.at[src64_v], rows_v.at[pl.ds(0, _C)], sem1)
            cpd.wait()
            cp2 = pltpu.async_copy(q_hbm.at[dst64_v], rows_v.at[pl.ds(_C, _C)], sem2)
            cp1.wait()
            cp2.wait()

            def _alpha(i, _):
                l1 = [_bfly(rows_v[i, pl.ds(h * 16, 16)]
                            * rows_v[_C + i, pl.ds(h * 16, 16)], 8)
                      for h in range(heads)]
                m1 = [jnp.where(idx16 < 8, l1[2 * j], l1[2 * j + 1]) for j in range(4)]
                l2 = [_bfly(m, 4) for m in m1]
                m2 = [jnp.where((idx16 & 4) == 0, l2[2 * j], l2[2 * j + 1]) for j in range(2)]
                l3 = [_bfly(m, 2) for m in m2]
                m3 = jnp.where((idx16 & 2) == 0, l3[0], l3[1])
                f = _bfly(m3, 1)
                vals64_v[i] = jnp.exp(f) * emask
                return _
            lax.fori_loop(0, _C, _alpha, 0)

            cpw = pltpu.async_copy(vals64_v, s_hbm.at[pl.ds(base, _C)], sem1)
            pltpu.sync_copy(vals64_v, acc_sh.at[dst64_v], add=True)
            cpw.wait()
        return carry
    lax.fori_loop(0, iters_d, _den_chunk, 0)

    plsc.subcore_barrier()
    pltpu.sync_copy(acc_sh.at[pl.ds(r0, rows_per_tile)],
                    den_hbm.at[cid, pl.ds(r0, rows_per_tile)])

    for h in range(heads):
        _zero_own_rows()
        plsc.subcore_barrier()

        def _num_chunk(t, carry, h=h):
            g = w + t * 32

            @pl.when(g < nchunk_n)
            def _guarded():
                base = g * _CN
                cps = pltpu.async_copy(src_hbm.at[pl.ds(base, _CN)], src128_v, sem1)
                cpd = pltpu.async_copy(dst_hbm.at[pl.ds(base, _CN)], dst128_v, sem2)
                cpv = pltpu.async_copy(s_hbm.at[pl.ds(base, _CN)], vals128_v, sem2)
                cps.wait()
                cpg = pltpu.async_copy(v_hbm.at[src128_v], rows_v, sem1)
                cpd.wait()
                cpv.wait()
                cpg.wait()

                def _scale(i, _):
                    srow = vals128_v[i]
                    sc_v[i] = rows_v[i, pl.ds(h * 16, 16)] * srow[_LANE_OF_HEAD[h]]
                    return _
                lax.fori_loop(0, _CN, _scale, 0)

                pltpu.sync_copy(sc_v, acc_sh.at[dst128_v], add=True)
            return carry
        lax.fori_loop(0, iters_n, _num_chunk, 0)

        plsc.subcore_barrier()
        pltpu.sync_copy(acc_sh.at[pl.ds(r0, rows_per_tile)],
                        num_hbm.at[cid * heads + h, pl.ds(r0, rows_per_tile)])
        plsc.subcore_barrier()


def _edge_phase(k, q, v, src, dst, n, e, heads):
    nchunk_d = e // _C
    iters_d = (nchunk_d + 31) // 32
    nchunk_n = e // _CN
    iters_n = (nchunk_n + 31) // 32
    rows_per_tile = -(-(n // 16) // 8) * 8
    n_pad = rows_per_tile * 16
    zr = 368
    assert rows_per_tile % zr == 0
    mesh = plsc.VectorSubcoreMesh(core_axis_name="c", subcore_axis_name="s")
    body = functools.partial(_edge_body, n_pad, e, heads, rows_per_tile, zr,
                             iters_d, nchunk_d, iters_n, nchunk_n)
    f = pl.kernel(
        body,
        mesh=mesh,
        out_type=[
            jax.ShapeDtypeStruct((2, n_pad, 16), jnp.float32),
            jax.ShapeDtypeStruct((2 * heads, n_pad, 16), jnp.float32),
            jax.ShapeDtypeStruct((e, 16), jnp.float32),
        ],
        scratch_types=[
            pltpu.VMEM((_C,), jnp.int32),
            pltpu.VMEM((_C,), jnp.int32),
            pltpu.VMEM((_CN,), jnp.int32),
            pltpu.VMEM((_CN,), jnp.int32),
            pltpu.VMEM((_CN, 128), jnp.float32),
            pltpu.VMEM((_C, 16), jnp.float32),
            pltpu.VMEM((_CN, 16), jnp.float32),
            pltpu.VMEM((_CN, 16), jnp.float32),
            pltpu.VMEM((zr, 16), jnp.float32),
            pltpu.VMEM_SHARED((n_pad, 16), jnp.float32),
            pltpu.SemaphoreType.DMA,
            pltpu.SemaphoreType.DMA,
        ],
        compiler_params=pltpu.CompilerParams(use_tc_tiling_on_sc=False),
    )
    den, num, _s = f(k, q, src, dst, v)
    return den, num


def _post_body(num_ref, den_ref, x_ref, pos_ref, wa_ref, ba_ref, wn_ref,
               bn_ref, beta_ref, lng_ref, lnb_ref, o_ref):
    heads = num_ref.shape[0] // 2
    t = x_ref.shape[0]
    d = x_ref.shape[1]
    dh = d // heads
    num = num_ref[...]
    den = den_ref[...]
    nsum = num[:heads] + num[heads:]
    dsum = den[0] + den[1]
    aggh = jnp.concatenate([nsum[h] for h in range(heads)], axis=-1)
    drep = jnp.concatenate(
        [jnp.broadcast_to(dsum[:, l:l + 1], (t, dh)) for l in _LANE_OF_HEAD[:heads]],
        axis=-1)
    agg = aggh / (drep + 1e-16)
    g = 0.5 * agg * (1.0 + jnp.tanh(_SQRT_2_OVER_PI * (agg + 0.044715 * agg * agg * agg)))
    o = jnp.dot(g, wa_ref[...], preferred_element_type=jnp.float32) + ba_ref[...]
    beta = beta_ref[0, 0]
    o = beta * o + (1.0 - beta) * x_ref[...]
    hh = jnp.dot(o, wn_ref[...], preferred_element_type=jnp.float32) + bn_ref[...] + pos_ref[...]
    mu = jnp.mean(hh, axis=-1, keepdims=True)
    var = jnp.mean((hh - mu) ** 2, axis=-1, keepdims=True)
    o_ref[...] = (hh - mu) * jax.lax.rsqrt(var + 1e-12) * lng_ref[...] + lnb_ref[...]


def _post_stage(num, den, x, pos_table, Wa, ba, Wn, bn, beta, ln_g, ln_b, tile):
    n, d = x.shape
    hh2 = num.shape[0]
    grid = n // tile
    full = lambda i: (0, 0)
    row = lambda i: (i, 0)
    row3 = lambda i: (0, i, 0)
    return pl.pallas_call(
        _post_body,
        grid=(grid,),
        in_specs=[
            pl.BlockSpec((hh2, tile, 16), row3),
            pl.BlockSpec((2, tile, 16), row3),
            pl.BlockSpec((tile, d), row),
            pl.BlockSpec((tile, d), row),
            pl.BlockSpec((d, d), full),
            pl.BlockSpec((1, d), full),
            pl.BlockSpec((d, d), full),
            pl.BlockSpec((1, d), full),
            pl.BlockSpec((1, 1), full),
            pl.BlockSpec((1, d), full),
            pl.BlockSpec((1, d), full),
        ],
        out_specs=pl.BlockSpec((tile, d), row),
        out_shape=jax.ShapeDtypeStruct((n, d), jnp.float32),
    )(num, den, x, pos_table, Wa, ba, Wn, bn, beta, ln_g, ln_b)


def kernel(x, edge_index, Wk, bk, Wq, bq, Wv, bv, a_rel, m_rel, p_rel,
           Wa, ba, skip, Wn, bn, pos_table, ln_g, ln_b):
    n, d = x.shape
    heads, dh = a_rel.shape[0], a_rel.shape[1]
    e = edge_index.shape[1]

    scale = p_rel / jnp.sqrt(jnp.float32(dh))
    Wk2 = jnp.einsum('ihd,hde->ihe', Wk.reshape(d, heads, dh), a_rel)
    Wk2 = (Wk2 * scale[None, :, None]).reshape(d, d)
    bk2 = (jnp.einsum('hd,hde->he', bk.reshape(heads, dh), a_rel) * scale[:, None]).reshape(1, d)
    Wv2 = jnp.einsum('ihd,hde->ihe', Wv.reshape(d, heads, dh), m_rel).reshape(d, d)
    bv2 = jnp.einsum('hd,hde->he', bv.reshape(heads, dh), m_rel).reshape(1, d)

    k, q, v = _projections(x, Wk2, bk2, Wq, bq.reshape(1, d), Wv2, bv2,
                           tile=2000, heads=heads)

    src = edge_index[0].astype(jnp.int32)
    dst = edge_index[1].astype(jnp.int32)
    den, num = _edge_phase(k, q, v, src, dst, n, e, heads)

    beta = jax.nn.sigmoid(skip).reshape(1, 1)
    return _post_stage(num, den, x, pos_table, Wa, ba.reshape(1, d),
                       Wn, bn.reshape(1, d), beta, ln_g.reshape(1, d),
                       ln_b.reshape(1, d), tile=2000)
